# baseline (speedup 1.0000x reference)
import functools

import jax
import jax.numpy as jnp
from jax import lax
from jax.experimental import pallas as pl
from jax.experimental.pallas import tpu as pltpu
from jax.experimental.pallas import tpu_sc as plsc

_N = 10000
_D_IN = 128
_HID = 256
_HH = _HID // 2
_NG = 128
_NCLS = 10
_NSC = 2
_NT = 16
_HALF = _N // _NSC
_EBLK = 128
_ACC_ROWS = _HALF + 8
_WCH_D = 208

_mesh = plsc.VectorSubcoreMesh(core_axis_name="c", subcore_axis_name="s")


def _localize(dloc_v, c):
    nb = c * _HALF
    for j in range(_EBLK // 16):
        d = dloc_v[pl.ds(j * 16, 16)]
        m = (d >= nb) & (d < nb + _HALF)
        dloc_v[pl.ds(j * 16, 16)] = jnp.where(m, d - nb, _HALF)


def _deg_body(nblk, dstp, deg, dloc_v, ones_v, buf_v, acc):
    c = lax.axis_index("c")
    s = lax.axis_index("s")

    def fill(r, _):
        ones_v[pl.ds(r * 16, 16)] = jnp.full((16,), 1.0, jnp.float32)
        return 0

    lax.fori_loop(0, _EBLK // 16, fill, 0)

    def fillb(r, _):
        buf_v[pl.ds(r * 16, 16)] = jnp.zeros((16,), jnp.float32)
        return 0

    lax.fori_loop(0, _WCH_D // 16, fillb, 0)

    def zchunk(k, _):
        ci = s + _NT * k

        @pl.when(ci < 39)
        def _():
            pltpu.sync_copy(buf_v.at[pl.ds(0, _EBLK)],
                            acc.at[pl.ds(ci * _EBLK, _EBLK)])
        return 0

    lax.fori_loop(0, 3, zchunk, 0)

    @pl.when(s == 0)
    def _():
        pltpu.sync_copy(buf_v.at[pl.ds(0, 16)], acc.at[pl.ds(4992, 16)])

    plsc.subcore_barrier()

    ept = nblk * _EBLK

    def step(b, _):
        off = s * ept + b * _EBLK
        pltpu.sync_copy(dstp.at[pl.ds(off, _EBLK)], dloc_v)
        _localize(dloc_v, c)
        pltpu.sync_copy(ones_v, acc.at[dloc_v], add=True)
        return 0

    lax.fori_loop(0, nblk, step, 0)
    plsc.subcore_barrier()

    def wchunk(k, _):
        ci = s + _NT * k

        @pl.when(ci < 25)
        def _():
            lb = ci * 200
            pltpu.sync_copy(acc.at[pl.ds(lb, 200)], buf_v.at[pl.ds(0, 200)])
            pltpu.sync_copy(buf_v.at[pl.ds(0, 200)],
                            deg.at[pl.ds(c * _HALF + lb, 200)])
        return 0

    lax.fori_loop(0, 2, wchunk, 0)


def _agg_pass(nblk, hs, out, srcp, dstp, src_v, dst_v, rows_v, isem, gsem,
              ssem, acc, c, s):
    def zero(r, _):
        for k in range(_HH // 16):
            rows_v[0, r, pl.ds(k * 16, 16)] = jnp.zeros((16,), jnp.float32)
        return 0

    lax.fori_loop(0, _EBLK, zero, 0)

    def zchunk(k, _):
        ci = s + _NT * k

        @pl.when(ci < 39)
        def _():
            pltpu.sync_copy(rows_v.at[0], acc.at[pl.ds(ci * _EBLK, _EBLK)])
        return 0

    lax.fori_loop(0, 3, zchunk, 0)

    @pl.when(s == 0)
    def _():
        pltpu.sync_copy(rows_v.at[0, pl.ds(0, 16)],
                        acc.at[pl.ds(4992, 16)])

    plsc.subcore_barrier()

    ept = nblk * _EBLK

    def step(b, _):
        ibuf = b % 4
        jbuf = (b - 1) % 4
        kbuf = (b - 2) % 4
        gbuf = (b - 1) % 2
        qbuf = (b - 2) % 2

        @pl.when(b < nblk)
        def _():
            off = s * ept + b * _EBLK
            pltpu.async_copy(srcp.at[pl.ds(off, _EBLK)], src_v.at[ibuf],
                             isem.at[ibuf])
            pltpu.async_copy(dstp.at[pl.ds(off, _EBLK)], dst_v.at[ibuf],
                             isem.at[ibuf])

        @pl.when((b >= 1) & (b <= nblk))
        def _():
            pltpu.make_async_copy(srcp.at[pl.ds(0, _EBLK)],
                                  src_v.at[jbuf], isem.at[jbuf]).wait()
            pltpu.make_async_copy(dstp.at[pl.ds(0, _EBLK)],
                                  dst_v.at[jbuf], isem.at[jbuf]).wait()
            for j in range(_EBLK // 16):
                nb = c * _HALF
                d = dst_v[jbuf, pl.ds(j * 16, 16)]
                m = (d >= nb) & (d < nb + _HALF)
                dst_v[jbuf, pl.ds(j * 16, 16)] = jnp.where(m, d - nb, _HALF)

            @pl.when(b >= 3)
            def _():
                pltpu.make_async_copy(hs.at[pl.ds(0, _EBLK)],
                                      rows_v.at[gbuf], ssem.at[gbuf]).wait()

            pltpu.async_copy(hs.at[src_v.at[jbuf]], rows_v.at[gbuf],
                             gsem.at[gbuf])

        @pl.when(b >= 2)
        def _():
            pltpu.make_async_copy(hs.at[pl.ds(0, _EBLK)],
                                  rows_v.at[qbuf], gsem.at[qbuf]).wait()
            pltpu.async_copy(rows_v.at[qbuf], acc.at[dst_v.at[kbuf]],
                             ssem.at[qbuf], add=True)
        return 0

    lax.fori_loop(0, nblk + 2, step, 0)

    for q in range(2):
        pltpu.make_async_copy(hs.at[pl.ds(0, _EBLK)], rows_v.at[q],
                              ssem.at[q]).wait()
    plsc.subcore_barrier()

    def wchunk(k, _):
        ci = s + _NT * k

        @pl.when(ci < 39)
        def _():
            lb = ci * _EBLK
            pltpu.sync_copy(acc.at[pl.ds(lb, _EBLK)], rows_v.at[0])
            pltpu.sync_copy(rows_v.at[0],
                            out.at[pl.ds(c * _HALF + lb, _EBLK)])

        @pl.when(ci == 39)
        def _():
            pltpu.sync_copy(acc.at[pl.ds(4992, 8)],
                            rows_v.at[0, pl.ds(0, 8)])
            pltpu.sync_copy(rows_v.at[0, pl.ds(0, 8)],
                            out.at[pl.ds(c * _HALF + 4992, 8)])
        return 0

    lax.fori_loop(0, 3, wchunk, 0)
    plsc.subcore_barrier()


def _agg_body(nblk, hsA, hsB, srcp, dstp, outA, outB, src_v, dst_v, rows_v,
              isem, gsem, ssem, acc):
    c = lax.axis_index("c")
    s = lax.axis_index("s")
    _agg_pass(nblk, hsA, outA, srcp, dstp, src_v, dst_v, rows_v, isem, gsem,
              ssem, acc, c, s)
    _agg_pass(nblk, hsB, outB, srcp, dstp, src_v, dst_v, rows_v, isem, gsem,
              ssem, acc, c, s)


def _mm1_body(x_ref, dg_ref, w_ref, oa_ref, ob_ref):
    dis = lax.rsqrt(dg_ref[...] + 1.0)
    h = jnp.dot(x_ref[...], w_ref[...],
                preferred_element_type=jnp.float32) * dis
    oa_ref[...] = h[:, :_HH]
    ob_ref[...] = h[:, _HH:]


def _mid_body(aa_ref, ab_ref, ha_ref, hb_ref, dg_ref, w_ref, b_ref,
              oa_ref, ob_ref):
    dis = lax.rsqrt(dg_ref[...] + 1.0)
    ta = jnp.maximum((aa_ref[...] + ha_ref[...]) * dis + b_ref[:, :_HH], 0.0)
    tb = jnp.maximum((ab_ref[...] + hb_ref[...]) * dis + b_ref[:, _HH:], 0.0)
    h = (jnp.dot(ta, w_ref[:_HH, :], preferred_element_type=jnp.float32) +
         jnp.dot(tb, w_ref[_HH:, :], preferred_element_type=jnp.float32))
    h = h * dis
    oa_ref[...] = h[:, :_HH]
    ob_ref[...] = h[:, _HH:]


def _head_body(aa_ref, ab_ref, ha_ref, hb_ref, dg_ref, b3_ref, batch_ref,
               wl_ref, bl_ref, o_ref, sums_ref, cnt_ref):
    i = pl.program_id(0)
    rows = aa_ref.shape[0]
    dis = lax.rsqrt(dg_ref[...] + 1.0)
    ya = (aa_ref[...] + ha_ref[...]) * dis + b3_ref[:, :_HH]
    yb = (ab_ref[...] + hb_ref[...]) * dis + b3_ref[:, _HH:]
    bvec = batch_ref[0, 0, :]
    onehot = (bvec[None, :] ==
              lax.broadcasted_iota(jnp.int32, (_NG, rows), 0)
              ).astype(jnp.float32)

    @pl.when(i == 0)
    def _():
        sums_ref[...] = jnp.zeros_like(sums_ref)
        cnt_ref[...] = jnp.zeros_like(cnt_ref)

    sums_ref[:, :_HH] += jnp.dot(onehot, ya,
                                 preferred_element_type=jnp.float32)
    sums_ref[:, _HH:] += jnp.dot(onehot, yb,
                                 preferred_element_type=jnp.float32)
    cnt_ref[...] += jnp.sum(onehot, axis=1, keepdims=True)

    @pl.when(i == pl.num_programs(0) - 1)
    def _():
        pooled = sums_ref[...] / jnp.maximum(cnt_ref[...], 1.0)
        o_ref[...] = jnp.dot(pooled, wl_ref[...],
                             preferred_element_type=jnp.float32) + bl_ref[...]


def kernel(x, edge_index, batch, W1, b1, W2, b2, W3, b3, Wl, bl):
    src = edge_index[0].astype(jnp.int32)
    dst = edge_index[1].astype(jnp.int32)
    E = src.shape[0]
    nblk = -(-E // (_NT * _EBLK))
    EP = _NT * nblk * _EBLK
    srcp = jnp.concatenate([src, jnp.zeros((EP - E,), jnp.int32)])
    dstp = jnp.concatenate([dst, jnp.full((EP - E,), _N, jnp.int32)])

    deg_call = pl.kernel(
        functools.partial(_deg_body, nblk),
        out_type=jax.ShapeDtypeStruct((_N,), jnp.float32),
        mesh=_mesh,
        scratch_types=[
            pltpu.VMEM((_EBLK,), jnp.int32),
            pltpu.VMEM((_EBLK,), jnp.float32),
            pltpu.VMEM((_WCH_D,), jnp.float32),
            pltpu.VMEM_SHARED((_ACC_ROWS,), jnp.float32),
        ],
    )
    degw = deg_call(dstp).reshape(_N, 1)

    agg_call = pl.kernel(
        functools.partial(_agg_body, nblk),
        out_type=(jax.ShapeDtypeStruct((_N, _HH), jnp.float32),
                  jax.ShapeDtypeStruct((_N, _HH), jnp.float32)),
        mesh=_mesh,
        scratch_types=[
            pltpu.VMEM((4, _EBLK), jnp.int32),
            pltpu.VMEM((4, _EBLK), jnp.int32),
            pltpu.VMEM((2, _EBLK, _HH), jnp.float32),
            pltpu.SemaphoreType.DMA((4,)),
            pltpu.SemaphoreType.DMA((2,)),
            pltpu.SemaphoreType.DMA((2,)),
            pltpu.VMEM_SHARED((_ACC_ROWS, _HH), jnp.float32),
        ],
    )

    rb = 1000
    grid = _N // rb
    hs_shapes = (jax.ShapeDtypeStruct((_N, _HH), jnp.float32),
                 jax.ShapeDtypeStruct((_N, _HH), jnp.float32))
    half_spec = pl.BlockSpec((rb, _HH), lambda i: (i, 0))
    mm1 = pl.pallas_call(
        _mm1_body,
        grid=(grid,),
        in_specs=[
            pl.BlockSpec((rb, _D_IN), lambda i: (i, 0)),
            pl.BlockSpec((rb, 1), lambda i: (i, 0)),
            pl.BlockSpec((_D_IN, _HID), lambda i: (0, 0)),
        ],
        out_specs=(half_spec, half_spec),
        out_shape=hs_shapes,
    )

    mid = pl.pallas_call(
        _mid_body,
        grid=(grid,),
        in_specs=[
            half_spec, half_spec, half_spec, half_spec,
            pl.BlockSpec((rb, 1), lambda i: (i, 0)),
            pl.BlockSpec((_HID, _HID), lambda i: (0, 0)),
            pl.BlockSpec((1, _HID), lambda i: (0, 0)),
        ],
        out_specs=(half_spec, half_spec),
        out_shape=hs_shapes,
    )

    hb = 2000
    hgrid = _N // hb
    hhalf_spec = pl.BlockSpec((hb, _HH), lambda i: (i, 0))
    head = pl.pallas_call(
        _head_body,
        grid=(hgrid,),
        in_specs=[
            hhalf_spec, hhalf_spec, hhalf_spec, hhalf_spec,
            pl.BlockSpec((hb, 1), lambda i: (i, 0)),
            pl.BlockSpec((1, _HID), lambda i: (0, 0)),
            pl.BlockSpec((1, 1, hb), lambda i: (i, 0, 0)),
            pl.BlockSpec((_HID, _NCLS), lambda i: (0, 0)),
            pl.BlockSpec((1, _NCLS), lambda i: (0, 0)),
        ],
        out_specs=pl.BlockSpec((_NG, _NCLS), lambda i: (0, 0)),
        out_shape=jax.ShapeDtypeStruct((_NG, _NCLS), jnp.float32),
        scratch_shapes=[
            pltpu.VMEM((_NG, _HID), jnp.float32),
            pltpu.VMEM((_NG, 1), jnp.float32),
        ],
    )

    batch3 = batch.astype(jnp.int32).reshape(hgrid, 1, hb)
    b1r = b1.reshape(1, _HID)
    b2r = b2.reshape(1, _HID)
    b3r = b3.reshape(1, _HID)
    blr = bl.reshape(1, _NCLS)

    h1a, h1b = mm1(x, degw, W1)
    a1a, a1b = agg_call(h1a, h1b, srcp, dstp)
    h2a, h2b = mid(a1a, a1b, h1a, h1b, degw, W2, b1r)
    a2a, a2b = agg_call(h2a, h2b, srcp, dstp)
    h3a, h3b = mid(a2a, a2b, h2a, h2b, degw, W3, b2r)
    a3a, a3b = agg_call(h3a, h3b, srcp, dstp)
    return head(a3a, a3b, h3a, h3b, degw, b3r, batch3, Wl, blr)


# speedup vs baseline: 1.0008x; 1.0008x over previous
import functools

import jax
import jax.numpy as jnp
from jax import lax
from jax.experimental import pallas as pl
from jax.experimental.pallas import tpu as pltpu
from jax.experimental.pallas import tpu_sc as plsc

_N = 10000
_D_IN = 128
_HID = 256
_HH = _HID // 2
_NG = 128
_NCLS = 10
_NSC = 2
_NT = 16
_HALF = _N // _NSC
_EBLK = 128
_ACC_ROWS = _HALF + 8
_WCH_D = 208

_mesh = plsc.VectorSubcoreMesh(core_axis_name="c", subcore_axis_name="s")


def _localize(dloc_v, c):
    nb = c * _HALF
    for j in range(_EBLK // 16):
        d = dloc_v[pl.ds(j * 16, 16)]
        m = (d >= nb) & (d < nb + _HALF)
        dloc_v[pl.ds(j * 16, 16)] = jnp.where(m, d - nb, _HALF)


def _deg_body(nblk, dstp, deg, dloc_v, ones_v, buf_v, isem, ssem, acc):
    c = lax.axis_index("c")
    s = lax.axis_index("s")

    def fill(r, _):
        ones_v[pl.ds(r * 16, 16)] = jnp.full((16,), 1.0, jnp.float32)
        return 0

    lax.fori_loop(0, _EBLK // 16, fill, 0)

    def fillb(r, _):
        buf_v[pl.ds(r * 16, 16)] = jnp.zeros((16,), jnp.float32)
        return 0

    lax.fori_loop(0, _WCH_D // 16, fillb, 0)

    def zchunk(k, _):
        ci = s + _NT * k

        @pl.when(ci < 39)
        def _():
            pltpu.sync_copy(buf_v.at[pl.ds(0, _EBLK)],
                            acc.at[pl.ds(ci * _EBLK, _EBLK)])
        return 0

    lax.fori_loop(0, 3, zchunk, 0)

    @pl.when(s == 0)
    def _():
        pltpu.sync_copy(buf_v.at[pl.ds(0, 16)], acc.at[pl.ds(4992, 16)])

    plsc.subcore_barrier()

    ept = nblk * _EBLK

    def step(b, _):
        ibuf = b % 4
        jbuf = (b - 1) % 4

        @pl.when(b < nblk)
        def _():
            @pl.when(b >= 4)
            def _():
                pltpu.make_async_copy(dstp.at[pl.ds(0, _EBLK)],
                                      dloc_v.at[ibuf], ssem.at[ibuf]).wait()
            off = s * ept + b * _EBLK
            pltpu.async_copy(dstp.at[pl.ds(off, _EBLK)], dloc_v.at[ibuf],
                             isem.at[ibuf])

        @pl.when(b >= 1)
        def _():
            pltpu.make_async_copy(dstp.at[pl.ds(0, _EBLK)],
                                  dloc_v.at[jbuf], isem.at[jbuf]).wait()
            for j in range(_EBLK // 16):
                nb = c * _HALF
                d = dloc_v[jbuf, pl.ds(j * 16, 16)]
                m = (d >= nb) & (d < nb + _HALF)
                dloc_v[jbuf, pl.ds(j * 16, 16)] = jnp.where(m, d - nb, _HALF)
            pltpu.async_copy(ones_v, acc.at[dloc_v.at[jbuf]],
                             ssem.at[jbuf], add=True)
        return 0

    lax.fori_loop(0, nblk + 1, step, 0)

    for q in range(4):
        pltpu.make_async_copy(dstp.at[pl.ds(0, _EBLK)],
                              dloc_v.at[q], ssem.at[q]).wait()

    plsc.subcore_barrier()

    def wchunk(k, _):
        ci = s + _NT * k

        @pl.when(ci < 25)
        def _():
            lb = ci * 200
            pltpu.sync_copy(acc.at[pl.ds(lb, 200)], buf_v.at[pl.ds(0, 200)])
            pltpu.sync_copy(buf_v.at[pl.ds(0, 200)],
                            deg.at[pl.ds(c * _HALF + lb, 200)])
        return 0

    lax.fori_loop(0, 2, wchunk, 0)


def _agg_pass(nblk, hs, out, srcp, dstp, src_v, dst_v, rows_v, isem, gsem,
              ssem, acc, c, s):
    def zero(r, _):
        for k in range(_HH // 16):
            rows_v[0, r, pl.ds(k * 16, 16)] = jnp.zeros((16,), jnp.float32)
        return 0

    lax.fori_loop(0, _EBLK, zero, 0)

    def zchunk(k, _):
        ci = s + _NT * k

        @pl.when(ci < 39)
        def _():
            pltpu.sync_copy(rows_v.at[0], acc.at[pl.ds(ci * _EBLK, _EBLK)])
        return 0

    lax.fori_loop(0, 3, zchunk, 0)

    @pl.when(s == 0)
    def _():
        pltpu.sync_copy(rows_v.at[0, pl.ds(0, 16)],
                        acc.at[pl.ds(4992, 16)])

    plsc.subcore_barrier()

    ept = nblk * _EBLK

    def step(b, _):
        ibuf = b % 4
        jbuf = (b - 1) % 4
        kbuf = (b - 2) % 4
        gbuf = (b - 1) % 2
        qbuf = (b - 2) % 2

        @pl.when(b < nblk)
        def _():
            off = s * ept + b * _EBLK
            pltpu.async_copy(srcp.at[pl.ds(off, _EBLK)], src_v.at[ibuf],
                             isem.at[ibuf])
            pltpu.async_copy(dstp.at[pl.ds(off, _EBLK)], dst_v.at[ibuf],
                             isem.at[ibuf])

        @pl.when((b >= 1) & (b <= nblk))
        def _():
            pltpu.make_async_copy(srcp.at[pl.ds(0, _EBLK)],
                                  src_v.at[jbuf], isem.at[jbuf]).wait()
            pltpu.make_async_copy(dstp.at[pl.ds(0, _EBLK)],
                                  dst_v.at[jbuf], isem.at[jbuf]).wait()
            for j in range(_EBLK // 16):
                nb = c * _HALF
                d = dst_v[jbuf, pl.ds(j * 16, 16)]
                m = (d >= nb) & (d < nb + _HALF)
                dst_v[jbuf, pl.ds(j * 16, 16)] = jnp.where(m, d - nb, _HALF)

            @pl.when(b >= 3)
            def _():
                pltpu.make_async_copy(hs.at[pl.ds(0, _EBLK)],
                                      rows_v.at[gbuf], ssem.at[gbuf]).wait()

            pltpu.async_copy(hs.at[src_v.at[jbuf]], rows_v.at[gbuf],
                             gsem.at[gbuf])

        @pl.when(b >= 2)
        def _():
            pltpu.make_async_copy(hs.at[pl.ds(0, _EBLK)],
                                  rows_v.at[qbuf], gsem.at[qbuf]).wait()
            pltpu.async_copy(rows_v.at[qbuf], acc.at[dst_v.at[kbuf]],
                             ssem.at[qbuf], add=True)
        return 0

    lax.fori_loop(0, nblk + 2, step, 0)

    for q in range(2):
        pltpu.make_async_copy(hs.at[pl.ds(0, _EBLK)], rows_v.at[q],
                              ssem.at[q]).wait()
    plsc.subcore_barrier()

    def wchunk(k, _):
        ci = s + _NT * k

        @pl.when(ci < 39)
        def _():
            lb = ci * _EBLK
            pltpu.sync_copy(acc.at[pl.ds(lb, _EBLK)], rows_v.at[0])
            pltpu.sync_copy(rows_v.at[0],
                            out.at[pl.ds(c * _HALF + lb, _EBLK)])

        @pl.when(ci == 39)
        def _():
            pltpu.sync_copy(acc.at[pl.ds(4992, 8)],
                            rows_v.at[0, pl.ds(0, 8)])
            pltpu.sync_copy(rows_v.at[0, pl.ds(0, 8)],
                            out.at[pl.ds(c * _HALF + 4992, 8)])
        return 0

    lax.fori_loop(0, 3, wchunk, 0)
    plsc.subcore_barrier()


def _agg_body(nblk, hsA, hsB, srcp, dstp, outA, outB, src_v, dst_v, rows_v,
              isem, gsem, ssem, acc):
    c = lax.axis_index("c")
    s = lax.axis_index("s")
    _agg_pass(nblk, hsA, outA, srcp, dstp, src_v, dst_v, rows_v, isem, gsem,
              ssem, acc, c, s)
    _agg_pass(nblk, hsB, outB, srcp, dstp, src_v, dst_v, rows_v, isem, gsem,
              ssem, acc, c, s)


def _mm1_body(x_ref, dg_ref, w_ref, oa_ref, ob_ref):
    dis = lax.rsqrt(dg_ref[...] + 1.0)
    h = jnp.dot(x_ref[...], w_ref[...],
                preferred_element_type=jnp.float32) * dis
    oa_ref[...] = h[:, :_HH]
    ob_ref[...] = h[:, _HH:]


def _mid_body(aa_ref, ab_ref, ha_ref, hb_ref, dg_ref, w_ref, b_ref,
              oa_ref, ob_ref):
    dis = lax.rsqrt(dg_ref[...] + 1.0)
    ta = jnp.maximum((aa_ref[...] + ha_ref[...]) * dis + b_ref[:, :_HH], 0.0)
    tb = jnp.maximum((ab_ref[...] + hb_ref[...]) * dis + b_ref[:, _HH:], 0.0)
    h = (jnp.dot(ta, w_ref[:_HH, :], preferred_element_type=jnp.float32) +
         jnp.dot(tb, w_ref[_HH:, :], preferred_element_type=jnp.float32))
    h = h * dis
    oa_ref[...] = h[:, :_HH]
    ob_ref[...] = h[:, _HH:]


def _head_body(aa_ref, ab_ref, ha_ref, hb_ref, dg_ref, b3_ref, batch_ref,
               wl_ref, bl_ref, o_ref, sums_ref, cnt_ref):
    i = pl.program_id(0)
    rows = aa_ref.shape[0]
    dis = lax.rsqrt(dg_ref[...] + 1.0)
    ya = (aa_ref[...] + ha_ref[...]) * dis + b3_ref[:, :_HH]
    yb = (ab_ref[...] + hb_ref[...]) * dis + b3_ref[:, _HH:]
    bvec = batch_ref[0, 0, :]
    onehot = (bvec[None, :] ==
              lax.broadcasted_iota(jnp.int32, (_NG, rows), 0)
              ).astype(jnp.float32)

    @pl.when(i == 0)
    def _():
        sums_ref[...] = jnp.zeros_like(sums_ref)
        cnt_ref[...] = jnp.zeros_like(cnt_ref)

    sums_ref[:, :_HH] += jnp.dot(onehot, ya,
                                 preferred_element_type=jnp.float32)
    sums_ref[:, _HH:] += jnp.dot(onehot, yb,
                                 preferred_element_type=jnp.float32)
    cnt_ref[...] += jnp.sum(onehot, axis=1, keepdims=True)

    @pl.when(i == pl.num_programs(0) - 1)
    def _():
        pooled = sums_ref[...] / jnp.maximum(cnt_ref[...], 1.0)
        o_ref[...] = jnp.dot(pooled, wl_ref[...],
                             preferred_element_type=jnp.float32) + bl_ref[...]


def kernel(x, edge_index, batch, W1, b1, W2, b2, W3, b3, Wl, bl):
    src = edge_index[0].astype(jnp.int32)
    dst = edge_index[1].astype(jnp.int32)
    E = src.shape[0]
    nblk = -(-E // (_NT * _EBLK))
    EP = _NT * nblk * _EBLK
    srcp = jnp.concatenate([src, jnp.zeros((EP - E,), jnp.int32)])
    dstp = jnp.concatenate([dst, jnp.full((EP - E,), _N, jnp.int32)])

    deg_call = pl.kernel(
        functools.partial(_deg_body, nblk),
        out_type=jax.ShapeDtypeStruct((_N,), jnp.float32),
        mesh=_mesh,
        scratch_types=[
            pltpu.VMEM((4, _EBLK), jnp.int32),
            pltpu.VMEM((_EBLK,), jnp.float32),
            pltpu.VMEM((_WCH_D,), jnp.float32),
            pltpu.SemaphoreType.DMA((4,)),
            pltpu.SemaphoreType.DMA((4,)),
            pltpu.VMEM_SHARED((_ACC_ROWS,), jnp.float32),
        ],
    )
    degw = deg_call(dstp).reshape(_N, 1)

    agg_call = pl.kernel(
        functools.partial(_agg_body, nblk),
        out_type=(jax.ShapeDtypeStruct((_N, _HH), jnp.float32),
                  jax.ShapeDtypeStruct((_N, _HH), jnp.float32)),
        mesh=_mesh,
        scratch_types=[
            pltpu.VMEM((4, _EBLK), jnp.int32),
            pltpu.VMEM((4, _EBLK), jnp.int32),
            pltpu.VMEM((2, _EBLK, _HH), jnp.float32),
            pltpu.SemaphoreType.DMA((4,)),
            pltpu.SemaphoreType.DMA((2,)),
            pltpu.SemaphoreType.DMA((2,)),
            pltpu.VMEM_SHARED((_ACC_ROWS, _HH), jnp.float32),
        ],
    )

    rb = 1000
    grid = _N // rb
    hs_shapes = (jax.ShapeDtypeStruct((_N, _HH), jnp.float32),
                 jax.ShapeDtypeStruct((_N, _HH), jnp.float32))
    half_spec = pl.BlockSpec((rb, _HH), lambda i: (i, 0))
    mm1 = pl.pallas_call(
        _mm1_body,
        grid=(grid,),
        in_specs=[
            pl.BlockSpec((rb, _D_IN), lambda i: (i, 0)),
            pl.BlockSpec((rb, 1), lambda i: (i, 0)),
            pl.BlockSpec((_D_IN, _HID), lambda i: (0, 0)),
        ],
        out_specs=(half_spec, half_spec),
        out_shape=hs_shapes,
    )

    mid = pl.pallas_call(
        _mid_body,
        grid=(grid,),
        in_specs=[
            half_spec, half_spec, half_spec, half_spec,
            pl.BlockSpec((rb, 1), lambda i: (i, 0)),
            pl.BlockSpec((_HID, _HID), lambda i: (0, 0)),
            pl.BlockSpec((1, _HID), lambda i: (0, 0)),
        ],
        out_specs=(half_spec, half_spec),
        out_shape=hs_shapes,
    )

    hb = 2000
    hgrid = _N // hb
    hhalf_spec = pl.BlockSpec((hb, _HH), lambda i: (i, 0))
    head = pl.pallas_call(
        _head_body,
        grid=(hgrid,),
        in_specs=[
            hhalf_spec, hhalf_spec, hhalf_spec, hhalf_spec,
            pl.BlockSpec((hb, 1), lambda i: (i, 0)),
            pl.BlockSpec((1, _HID), lambda i: (0, 0)),
            pl.BlockSpec((1, 1, hb), lambda i: (i, 0, 0)),
            pl.BlockSpec((_HID, _NCLS), lambda i: (0, 0)),
            pl.BlockSpec((1, _NCLS), lambda i: (0, 0)),
        ],
        out_specs=pl.BlockSpec((_NG, _NCLS), lambda i: (0, 0)),
        out_shape=jax.ShapeDtypeStruct((_NG, _NCLS), jnp.float32),
        scratch_shapes=[
            pltpu.VMEM((_NG, _HID), jnp.float32),
            pltpu.VMEM((_NG, 1), jnp.float32),
        ],
    )

    batch3 = batch.astype(jnp.int32).reshape(hgrid, 1, hb)
    b1r = b1.reshape(1, _HID)
    b2r = b2.reshape(1, _HID)
    b3r = b3.reshape(1, _HID)
    blr = bl.reshape(1, _NCLS)

    h1a, h1b = mm1(x, degw, W1)
    a1a, a1b = agg_call(h1a, h1b, srcp, dstp)
    h2a, h2b = mid(a1a, a1b, h1a, h1b, degw, W2, b1r)
    a2a, a2b = agg_call(h2a, h2b, srcp, dstp)
    h3a, h3b = mid(a2a, a2b, h2a, h2b, degw, W3, b2r)
    a3a, a3b = agg_call(h3a, h3b, srcp, dstp)
    return head(a3a, a3b, h3a, h3b, degw, b3r, batch3, Wl, blr)
